# Initial kernel scaffold; baseline (speedup 1.0000x reference)
import jax
import jax.numpy as jnp
from jax.experimental import pallas as pl


def kernel(h, mps_edge_index_0, mps_edge_weight_0, mps_edge_index_1, mps_edge_weight_1, W0, b0, W1, b1, fc_W, fc_b, att):
    raise NotImplementedError("write your pallas kernel here")


# speedup vs baseline: 4.1609x; 4.1609x over previous
import functools
import jax
import jax.numpy as jnp
from jax import lax
from jax.experimental import pallas as pl
from jax.experimental.pallas import tpu as pltpu, tpu_sc as plsc

N = 10000
D = 128
H = 128
E = 320000

NS = 16
CHUNK = 128
CHUNKS_PER_TILE = 157
EP_TILE = CHUNKS_PER_TILE * CHUNK
EP = EP_TILE * NS
WB_TILES = 10
ROWS_PER_TILE = N // WB_TILES


def _pre_body(h_ref, w_ref, out_ref):
    out_ref[...] = jnp.tanh(
        jax.lax.dot_general(h_ref[...], w_ref[0],
                            (((1,), (1,)), ((), ())),
                            preferred_element_type=jnp.float32))


def _make_table(h, W0, W1):
    blk = 1000
    nblk = N // blk
    Ws = jnp.stack([W0, W1])
    return pl.pallas_call(
        _pre_body,
        grid=(2, nblk),
        in_specs=[
            pl.BlockSpec((blk, D), lambda m, i: (i, 0)),
            pl.BlockSpec((1, H, D), lambda m, i: (m, 0, 0)),
        ],
        out_specs=pl.BlockSpec((blk, H), lambda m, i: (m * nblk + i, 0)),
        out_shape=jax.ShapeDtypeStruct((2 * N, H), jnp.float32),
    )(h, Ws)


def _sc_body(table, cols, rows, ews, binit, e0, e1,
             idx_v, ridx_v, ew_v, msg_v, acc, sem):
    c = lax.axis_index("c")
    s = lax.axis_index("s")

    @pl.when(s < WB_TILES)
    def _():
        pltpu.sync_copy(binit.at[c],
                        acc.at[pl.ds(s * ROWS_PER_TILE, ROWS_PER_TILE)])
    plsc.subcore_barrier()

    def chunk_body(k, _):
        base = (s * CHUNKS_PER_TILE + k) * CHUNK
        pltpu.sync_copy(cols.at[c, pl.ds(base, CHUNK)], idx_v)
        pltpu.sync_copy(rows.at[c, pl.ds(base, CHUNK)], ridx_v)
        pltpu.sync_copy(ews.at[c, pl.ds(base, CHUNK)], ew_v)
        pltpu.async_copy(table.at[idx_v], msg_v, sem).wait()

        dnums = lax.GatherDimensionNumbers(
            offset_dims=(), collapsed_slice_dims=(0,), start_index_map=(0,))

        def grp_body(g, _):
            ew16 = ew_v[pl.ds(g * 16, 16)]
            for j in range(16):
                e = g * 16 + j
                w = lax.gather(ew16, jnp.full((16, 1), j, jnp.int32), dnums,
                               (1,),
                               mode=lax.GatherScatterMode.PROMISE_IN_BOUNDS)
                for f in range(H // 16):
                    sl = pl.ds(f * 16, 16)
                    msg_v[e, sl] = msg_v[e, sl] * w
            return 0

        lax.fori_loop(0, CHUNK // 16, grp_body, 0)
        pltpu.sync_copy(msg_v, acc.at[ridx_v], add=True)
        return 0

    lax.fori_loop(0, CHUNKS_PER_TILE, chunk_body, 0)
    plsc.subcore_barrier()

    sl = pl.ds(s * ROWS_PER_TILE, ROWS_PER_TILE)

    @pl.when(jnp.logical_and(c == 0, s < WB_TILES))
    def _():
        pltpu.sync_copy(acc.at[sl], e0.at[sl])

    @pl.when(jnp.logical_and(c == 1, s < WB_TILES))
    def _():
        pltpu.sync_copy(acc.at[sl], e1.at[sl])


def _segment_spmm(table, cols, rows, ews, binit):
    mesh = plsc.VectorSubcoreMesh(core_axis_name="c", subcore_axis_name="s")
    f = functools.partial(
        pl.kernel,
        out_type=(jax.ShapeDtypeStruct((N, H), jnp.float32),
                  jax.ShapeDtypeStruct((N, H), jnp.float32)),
        mesh=mesh,
        scratch_types=[
            pltpu.VMEM((CHUNK,), jnp.int32),
            pltpu.VMEM((CHUNK,), jnp.int32),
            pltpu.VMEM((CHUNK,), jnp.float32),
            pltpu.VMEM((CHUNK, H), jnp.float32),
            pltpu.VMEM_SHARED((N, H), jnp.float32),
            pltpu.SemaphoreType.DMA,
        ],
    )(_sc_body)
    return f(table, cols, rows, ews, binit)


def _sums_body(e0_ref, e1_ref, fcw_ref, fcb_ref, out_ref):
    i = pl.program_id(0)
    dn = (((1,), (1,)), ((), ()))
    t0 = jnp.tanh(jax.lax.dot_general(e0_ref[...], fcw_ref[...], dn,
                                      preferred_element_type=jnp.float32)
                  + fcb_ref[...])
    t1 = jnp.tanh(jax.lax.dot_general(e1_ref[...], fcw_ref[...], dn,
                                      preferred_element_type=jnp.float32)
                  + fcb_ref[...])
    part = jnp.stack([jnp.sum(t0, axis=0), jnp.sum(t1, axis=0)])

    @pl.when(i == 0)
    def _():
        out_ref[...] = part

    @pl.when(i > 0)
    def _():
        out_ref[...] = out_ref[...] + part


def _attn_sums(e0, e1, fc_W, fc_b):
    blk = 1000
    nblk = N // blk
    return pl.pallas_call(
        _sums_body,
        grid=(nblk,),
        in_specs=[
            pl.BlockSpec((blk, H), lambda i: (i, 0)),
            pl.BlockSpec((blk, H), lambda i: (i, 0)),
            pl.BlockSpec((H, H), lambda i: (0, 0)),
            pl.BlockSpec((1, H), lambda i: (0, 0)),
        ],
        out_specs=pl.BlockSpec((2, H), lambda i: (0, 0)),
        out_shape=jax.ShapeDtypeStruct((2, H), jnp.float32),
    )(e0, e1, fc_W, fc_b.reshape(1, H))


def _combine_body(sums_ref, att_ref, e0_ref, e1_ref, z_ref):
    sp = sums_ref[...] * (1.0 / N)
    logits = jnp.sum(att_ref[...] * sp, axis=1)
    m = jnp.max(logits)
    ex = jnp.exp(logits - m)
    beta = ex / jnp.sum(ex)
    z_ref[...] = beta[0] * e0_ref[...] + beta[1] * e1_ref[...]


def _combine(sums, att, e0, e1):
    blk = 1000
    nblk = N // blk
    return pl.pallas_call(
        _combine_body,
        grid=(nblk,),
        in_specs=[
            pl.BlockSpec((2, H), lambda i: (0, 0)),
            pl.BlockSpec((1, H), lambda i: (0, 0)),
            pl.BlockSpec((blk, H), lambda i: (i, 0)),
            pl.BlockSpec((blk, H), lambda i: (i, 0)),
        ],
        out_specs=pl.BlockSpec((blk, H), lambda i: (i, 0)),
        out_shape=jax.ShapeDtypeStruct((N, H), jnp.float32),
    )(sums, att, e0, e1)


def _pad_edges(col, row, ew):
    pad = EP - E
    col = jnp.concatenate([col, jnp.zeros((pad,), col.dtype)])
    row = jnp.concatenate([row, jnp.zeros((pad,), row.dtype)])
    ew = jnp.concatenate([ew, jnp.zeros((pad,), ew.dtype)])
    return col, row, ew


def kernel(h, mps_edge_index_0, mps_edge_weight_0,
           mps_edge_index_1, mps_edge_weight_1,
           W0, b0, W1, b1, fc_W, fc_b, att):
    col0 = mps_edge_index_0[1].astype(jnp.int32)
    row0 = mps_edge_index_0[0].astype(jnp.int32)
    col1 = mps_edge_index_1[1].astype(jnp.int32) + N
    row1 = mps_edge_index_1[0].astype(jnp.int32)
    c0, r0, w0 = _pad_edges(col0, row0, mps_edge_weight_0)
    c1, r1, w1 = _pad_edges(col1, row1, mps_edge_weight_1)
    cols = jnp.stack([c0, c1])
    rows = jnp.stack([r0, r1])
    ews = jnp.stack([w0, w1])
    binit = jnp.stack([
        jnp.broadcast_to(b0[None, :], (ROWS_PER_TILE, H)),
        jnp.broadcast_to(b1[None, :], (ROWS_PER_TILE, H)),
    ])

    table = _make_table(h, W0, W1)
    e0, e1 = _segment_spmm(table, cols, rows, ews, binit)
    sums = _attn_sums(e0, e1, fc_W, fc_b)
    z = _combine(sums, att, e0, e1)
    return (z, e0, e1)


# speedup vs baseline: 4.5814x; 1.1011x over previous
import functools
import jax
import jax.numpy as jnp
from jax import lax
from jax.experimental import pallas as pl
from jax.experimental.pallas import tpu as pltpu, tpu_sc as plsc

N = 10000
D = 128
H = 128
E = 320000

NS = 16
CHUNK = 128
RING = 3
STAGES = 159
EP_TILE = STAGES * CHUNK
EP = EP_TILE * NS
WB_TILES = 10
ROWS_PER_TILE = N // WB_TILES


def _pre_body(h_ref, w_ref, out_ref):
    out_ref[...] = jnp.tanh(
        jax.lax.dot_general(h_ref[...], w_ref[0],
                            (((1,), (1,)), ((), ())),
                            preferred_element_type=jnp.float32))


def _make_table(h, W0, W1):
    blk = 1000
    nblk = N // blk
    Ws = jnp.stack([W0, W1])
    return pl.pallas_call(
        _pre_body,
        grid=(2, nblk),
        in_specs=[
            pl.BlockSpec((blk, D), lambda m, i: (i, 0)),
            pl.BlockSpec((1, H, D), lambda m, i: (m, 0, 0)),
        ],
        out_specs=pl.BlockSpec((blk, H), lambda m, i: (m * nblk + i, 0)),
        out_shape=jax.ShapeDtypeStruct((2 * N, H), jnp.float32),
    )(h, Ws)


def _sc_body(table, packed, ews, binit, e0, e1,
             meta_v, ew_v, msg_v, acc, *sems):
    sg = sems[0:RING]
    ss = sems[RING:2 * RING]
    sm = sems[2 * RING:3 * RING]
    c = lax.axis_index("c")
    s = lax.axis_index("s")

    @pl.when(s < WB_TILES)
    def _():
        pltpu.sync_copy(binit.at[c],
                        acc.at[pl.ds(s * ROWS_PER_TILE, ROWS_PER_TILE)])
    plsc.subcore_barrier()

    dnums = lax.GatherDimensionNumbers(
        offset_dims=(), collapsed_slice_dims=(0,), start_index_map=(0,))

    def fire_meta(stage, u):
        pltpu.async_copy(packed.at[c, s, stage], meta_v.at[u], sm[u])
        pltpu.async_copy(ews.at[c, s, stage], ew_v.at[u], sm[u])

    def wait_meta(u):
        pltpu.make_async_copy(packed.at[c, s, 0], meta_v.at[u], sm[u]).wait()
        pltpu.make_async_copy(ews.at[c, s, 0], ew_v.at[u], sm[u]).wait()

    def fire_gather(u):
        pltpu.async_copy(table.at[meta_v.at[u, 0]], msg_v.at[u], sg[u])

    def wait_gather(u):
        pltpu.make_async_copy(table.at[meta_v.at[u, 0]], msg_v.at[u],
                              sg[u]).wait()

    def fire_scatter(u):
        pltpu.async_copy(msg_v.at[u], acc.at[meta_v.at[u, 1]], ss[u],
                         add=True)

    def wait_scatter(u):
        pltpu.make_async_copy(msg_v.at[u], acc.at[meta_v.at[u, 1]],
                              ss[u]).wait()

    for u in range(2):
        fire_meta(u, u)
    for u in range(2):
        wait_meta(u)
        fire_gather(u)

    def outer_body(i, _):
        for u in range(RING):
            k = i * RING + u
            up2 = (u + 2) % RING

            @pl.when(k > 0)
            def _():
                wait_scatter((u + RING - 1) % RING)

            @pl.when(k + 2 < STAGES)
            def _():
                fire_meta(k + 2, up2)

            wait_gather(u)

            def grp_body(g, _):
                ew16 = ew_v[u, pl.ds(g * 16, 16)]
                for j in range(16):
                    w = lax.gather(ew16, jnp.full((16, 1), j, jnp.int32),
                                   dnums, (1,),
                                   mode=lax.GatherScatterMode.PROMISE_IN_BOUNDS)
                    e = g * 16 + j
                    for f in range(H // 16):
                        sl = pl.ds(f * 16, 16)
                        msg_v[u, e, sl] = msg_v[u, e, sl] * w
                return 0

            lax.fori_loop(0, CHUNK // 16, grp_body, 0)
            fire_scatter(u)

            @pl.when(k + 2 < STAGES)
            def _():
                wait_meta(up2)
                fire_gather(up2)
        return 0

    lax.fori_loop(0, STAGES // RING, outer_body, 0)
    wait_scatter((STAGES - 1) % RING)
    plsc.subcore_barrier()

    sl = pl.ds(s * ROWS_PER_TILE, ROWS_PER_TILE)

    @pl.when(jnp.logical_and(c == 0, s < WB_TILES))
    def _():
        pltpu.sync_copy(acc.at[sl], e0.at[sl])

    @pl.when(jnp.logical_and(c == 1, s < WB_TILES))
    def _():
        pltpu.sync_copy(acc.at[sl], e1.at[sl])


def _segment_spmm(table, packed, ews, binit):
    mesh = plsc.VectorSubcoreMesh(core_axis_name="c", subcore_axis_name="s")
    f = functools.partial(
        pl.kernel,
        out_type=(jax.ShapeDtypeStruct((N, H), jnp.float32),
                  jax.ShapeDtypeStruct((N, H), jnp.float32)),
        mesh=mesh,
        scratch_types=[
            pltpu.VMEM((RING, 2, CHUNK), jnp.int32),
            pltpu.VMEM((RING, CHUNK), jnp.float32),
            pltpu.VMEM((RING, CHUNK, H), jnp.float32),
            pltpu.VMEM_SHARED((N, H), jnp.float32),
        ] + [pltpu.SemaphoreType.DMA] * (3 * RING),
    )(_sc_body)
    return f(table, packed, ews, binit)


def _sums_body(e0_ref, e1_ref, fcw_ref, fcb_ref, out_ref):
    i = pl.program_id(0)
    dn = (((1,), (1,)), ((), ()))
    t0 = jnp.tanh(jax.lax.dot_general(e0_ref[...], fcw_ref[...], dn,
                                      preferred_element_type=jnp.float32)
                  + fcb_ref[...])
    t1 = jnp.tanh(jax.lax.dot_general(e1_ref[...], fcw_ref[...], dn,
                                      preferred_element_type=jnp.float32)
                  + fcb_ref[...])
    part = jnp.stack([jnp.sum(t0, axis=0), jnp.sum(t1, axis=0)])

    @pl.when(i == 0)
    def _():
        out_ref[...] = part

    @pl.when(i > 0)
    def _():
        out_ref[...] = out_ref[...] + part


def _attn_sums(e0, e1, fc_W, fc_b):
    blk = 1000
    nblk = N // blk
    return pl.pallas_call(
        _sums_body,
        grid=(nblk,),
        in_specs=[
            pl.BlockSpec((blk, H), lambda i: (i, 0)),
            pl.BlockSpec((blk, H), lambda i: (i, 0)),
            pl.BlockSpec((H, H), lambda i: (0, 0)),
            pl.BlockSpec((1, H), lambda i: (0, 0)),
        ],
        out_specs=pl.BlockSpec((2, H), lambda i: (0, 0)),
        out_shape=jax.ShapeDtypeStruct((2, H), jnp.float32),
    )(e0, e1, fc_W, fc_b.reshape(1, H))


def _combine_body(sums_ref, att_ref, e0_ref, e1_ref, z_ref):
    sp = sums_ref[...] * (1.0 / N)
    logits = jnp.sum(att_ref[...] * sp, axis=1)
    m = jnp.max(logits)
    ex = jnp.exp(logits - m)
    beta = ex / jnp.sum(ex)
    z_ref[...] = beta[0] * e0_ref[...] + beta[1] * e1_ref[...]


def _combine(sums, att, e0, e1):
    blk = 1000
    nblk = N // blk
    return pl.pallas_call(
        _combine_body,
        grid=(nblk,),
        in_specs=[
            pl.BlockSpec((2, H), lambda i: (0, 0)),
            pl.BlockSpec((1, H), lambda i: (0, 0)),
            pl.BlockSpec((blk, H), lambda i: (i, 0)),
            pl.BlockSpec((blk, H), lambda i: (i, 0)),
        ],
        out_specs=pl.BlockSpec((blk, H), lambda i: (i, 0)),
        out_shape=jax.ShapeDtypeStruct((N, H), jnp.float32),
    )(sums, att, e0, e1)


def _pack_edges(col, row, ew):
    pad = EP - E
    col = jnp.concatenate([col, jnp.zeros((pad,), col.dtype)])
    row = jnp.concatenate([row, jnp.zeros((pad,), row.dtype)])
    ew = jnp.concatenate([ew, jnp.zeros((pad,), ew.dtype)])
    idx = jnp.stack([col.reshape(NS, STAGES, CHUNK),
                     row.reshape(NS, STAGES, CHUNK)], axis=2)
    return idx, ew.reshape(NS, STAGES, CHUNK)


def kernel(h, mps_edge_index_0, mps_edge_weight_0,
           mps_edge_index_1, mps_edge_weight_1,
           W0, b0, W1, b1, fc_W, fc_b, att):
    col0 = mps_edge_index_0[1].astype(jnp.int32)
    row0 = mps_edge_index_0[0].astype(jnp.int32)
    col1 = mps_edge_index_1[1].astype(jnp.int32) + N
    row1 = mps_edge_index_1[0].astype(jnp.int32)
    idx0, ewr0 = _pack_edges(col0, row0, mps_edge_weight_0)
    idx1, ewr1 = _pack_edges(col1, row1, mps_edge_weight_1)
    packed = jnp.stack([idx0, idx1])
    ews = jnp.stack([ewr0, ewr1])
    binit = jnp.stack([
        jnp.broadcast_to(b0[None, :], (ROWS_PER_TILE, H)),
        jnp.broadcast_to(b1[None, :], (ROWS_PER_TILE, H)),
    ])

    table = _make_table(h, W0, W1)
    e0, e1 = _segment_spmm(table, packed, ews, binit)
    sums = _attn_sums(e0, e1, fc_W, fc_b)
    z = _combine(sums, att, e0, e1)
    return (z, e0, e1)
